# Initial kernel scaffold; baseline (speedup 1.0000x reference)
import jax
import jax.numpy as jnp
from jax.experimental import pallas as pl


def kernel(x, edge_index, init_minhash, init_hll, W_enc, b_enc, W1, b1, W2, b2):
    raise NotImplementedError("write your pallas kernel here")


# speedup vs baseline: 8.7134x; 8.7134x over previous
import functools

import jax
import jax.numpy as jnp
from jax import lax
from jax.experimental import pallas as pl
from jax.experimental.pallas import tpu as pltpu
from jax.experimental.pallas import tpu_sc as plsc

L = 16
SB = 4096
_LN2 = 0.6931471805599453


def _row_ids(r):
    return jnp.full((L,), r, jnp.int32)


def _scan_blocks(src_hbm, dst_hbm, nblocks, sbuf, dbuf, chunk_fn):
    def blk(b, carry):
        off = pl.multiple_of(b * jnp.int32(SB), SB)
        pltpu.sync_copy(src_hbm.at[pl.ds(off, SB)], sbuf)
        pltpu.sync_copy(dst_hbm.at[pl.ds(off, SB)], dbuf)

        def chunk(j, c2):
            coff = pl.multiple_of(j * jnp.int32(L), L)
            s_idx = sbuf[pl.ds(coff, L)]
            d_idx = dbuf[pl.ds(coff, L)]
            chunk_fn(s_idx, d_idx)
            return c2
        return lax.fori_loop(jnp.int32(0), jnp.int32(SB // L), chunk, carry)
    lax.fori_loop(jnp.int32(0), jnp.int32(nblocks), blk, jnp.int32(0))


def _copy_rows(src, dst, nrows, np_):
    def b(i, carry):
        off = pl.multiple_of(i * jnp.int32(L), L)
        for r in range(nrows):
            dst[r, pl.ds(off, L)] = src[r, pl.ds(off, L)]
        return carry
    lax.fori_loop(jnp.int32(0), jnp.int32(np_ // L), b, jnp.int32(0))


def _zero_rows(dst, nrows, np_):
    z = jnp.zeros((L,), dst.dtype)
    def b(i, carry):
        off = pl.multiple_of(i * jnp.int32(L), L)
        for r in range(nrows):
            dst[r, pl.ds(off, L)] = z
        return carry
    lax.fori_loop(jnp.int32(0), jnp.int32(np_ // L), b, jnp.int32(0))


def _minmax_update(new_ref, row, s_vals, d_idx, is_min):
    def cond(m):
        return jnp.max(m.astype(jnp.int32)) > 0

    def body(m):
        cur = plsc.load_gather(new_ref, [row, d_idx])
        val = jnp.minimum(cur, s_vals) if is_min else jnp.maximum(cur, s_vals)
        plsc.store_scatter(new_ref, [row, d_idx], val, mask=m)
        chk = plsc.load_gather(new_ref, [row, d_idx])
        bad = (chk > s_vals) if is_min else (chk < s_vals)
        return m & bad

    lax.while_loop(cond, body, jnp.ones((L,), jnp.bool_))


def _sc_minmax_body(np_, nblocks,
                    mhT, hllT, srcE, dstE,
                    mh2T, hll1T, hll2T, degc,
                    old4, new4, sbuf, dbuf, degt):
    wid = lax.axis_index("s") * jnp.int32(2) + lax.axis_index("c")
    rows = [_row_ids(r) for r in range(4)]

    base = wid * jnp.int32(4)
    pltpu.sync_copy(mhT.at[pl.ds(base, 4)], old4)
    pltpu.sync_copy(mhT.at[pl.ds(base, 4)], new4)

    def mh_chunk(s_idx, d_idx):
        for r in range(4):
            sv = plsc.load_gather(old4, [rows[r], s_idx])
            _minmax_update(new4, rows[r], sv, d_idx, is_min=True)

    _scan_blocks(srcE, dstE, nblocks, sbuf, dbuf, mh_chunk)
    _copy_rows(new4, old4, 4, np_)
    _scan_blocks(srcE, dstE, nblocks, sbuf, dbuf, mh_chunk)
    pltpu.sync_copy(new4, mh2T.at[pl.ds(base, 4)])

    @pl.when(wid < 10)
    def _():
        pltpu.sync_copy(hllT.at[pl.ds(wid, 1)], old4.at[pl.ds(0, 1)])
        pltpu.sync_copy(hllT.at[pl.ds(wid, 1)], new4.at[pl.ds(0, 1)])

        def hl_chunk(s_idx, d_idx):
            sv = plsc.load_gather(old4, [rows[0], s_idx])
            _minmax_update(new4, rows[0], sv, d_idx, is_min=False)

        _scan_blocks(srcE, dstE, nblocks, sbuf, dbuf, hl_chunk)
        pltpu.sync_copy(new4.at[pl.ds(0, 1)], hll1T.at[pl.ds(wid, 1)])
        _copy_rows(new4, old4, 1, np_)
        _scan_blocks(srcE, dstE, nblocks, sbuf, dbuf, hl_chunk)
        pltpu.sync_copy(new4.at[pl.ds(0, 1)], hll2T.at[pl.ds(wid, 1)])

    @pl.when(wid == 10)
    def _():
        z = jnp.zeros((L,), jnp.int32)

        def zb(i, carry):
            degt[pl.ds(pl.multiple_of(i * jnp.int32(L), L), L)] = z
            return carry
        lax.fori_loop(jnp.int32(0), jnp.int32(np_ // L), zb, jnp.int32(0))

        ones = jnp.ones((L,), jnp.int32)

        def dg_chunk(s_idx, d_idx):
            plsc.addupdate_scatter(degt, [d_idx], ones)

        _scan_blocks(srcE, dstE, nblocks, sbuf, dbuf, dg_chunk)
        pltpu.sync_copy(degt, degc)


def _sc_gcn_body(np_, nblocks, gT, srcE, dstE, accT, val4, acc4, sbuf, dbuf):
    wid = lax.axis_index("s") * jnp.int32(2) + lax.axis_index("c")
    rows = [_row_ids(r) for r in range(4)]
    for p in range(2):
        base = wid * jnp.int32(8) + jnp.int32(p * 4)
        pltpu.sync_copy(gT.at[pl.ds(base, 4)], val4)
        _zero_rows(acc4, 4, np_)

        def chunk(s_idx, d_idx):
            for r in range(4):
                v = plsc.load_gather(val4, [rows[r], s_idx])
                plsc.addupdate_scatter(acc4, [rows[r], d_idx], v)

        _scan_blocks(srcE, dstE, nblocks, sbuf, dbuf, chunk)
        pltpu.sync_copy(acc4, accT.at[pl.ds(base, 4)])


def _tc_enc_body(xb, Wb, bb, ob):
    ob[...] = jnp.dot(xb[...], Wb[...],
                      preferred_element_type=jnp.float32) + bb[...]


def _tc_g1_body(hb, Wb, degb, gb, dvb):
    dinv = lax.rsqrt(degb[...].astype(jnp.float32) + 1.0)
    dvb[...] = dinv
    gb[...] = dinv * jnp.dot(hb[...], Wb[...],
                             preferred_element_type=jnp.float32)


def _tc_layer_body(hb, gb, ab, dvb, bb, Wb, h1b, g2b):
    h1 = hb[...] + dvb[...] * (ab[...] + gb[...]) + bb[...]
    h1b[...] = h1
    g2b[...] = dvb[...] * jnp.dot(h1, Wb[...],
                                  preferred_element_type=jnp.float32)


def _tc_final_body(alpha_m2, hb, gb, ab, dvb, bb, r1b, r2b,
                   hob, c1b, c2b):
    hob[...] = hb[...] + dvb[...] * (ab[...] + gb[...]) + bb[...]
    z1 = jnp.sum(jnp.exp(r1b[...].astype(jnp.float32) * (-_LN2)),
                 axis=1, keepdims=True)
    z2 = jnp.sum(jnp.exp(r2b[...].astype(jnp.float32) * (-_LN2)),
                 axis=1, keepdims=True)
    c1b[...] = alpha_m2 / z1
    c2b[...] = alpha_m2 / z2


def kernel(x, edge_index, init_minhash, init_hll, W_enc, b_enc, W1, b1, W2, b2):
    N, F = x.shape
    H = W_enc.shape[1]
    E = edge_index.shape[1]
    NPERM = init_minhash.shape[1]
    M = init_hll.shape[1]
    NP = ((N + L) // L) * L
    NB = -(-E // SB)
    EP = NB * SB

    i32 = jnp.int32
    f32 = jnp.float32

    src = jnp.concatenate(
        [edge_index[0].astype(i32), jnp.full((EP - E,), N, i32)])
    dst = jnp.concatenate(
        [edge_index[1].astype(i32), jnp.full((EP - E,), N, i32)])
    mhT = jnp.pad(init_minhash.astype(i32).T, ((0, 0), (0, NP - N)))
    hllT = jnp.pad(init_hll.astype(i32).T, ((0, 0), (0, NP - N)))

    mesh = plsc.VectorSubcoreMesh(core_axis_name="c", subcore_axis_name="s")

    minmax_call = pl.kernel(
        functools.partial(_sc_minmax_body, NP, NB),
        out_type=[
            jax.ShapeDtypeStruct((NPERM, NP), i32),
            jax.ShapeDtypeStruct((M, NP), i32),
            jax.ShapeDtypeStruct((M, NP), i32),
            jax.ShapeDtypeStruct((NP,), i32),
        ],
        mesh=mesh,
        compiler_params=pltpu.CompilerParams(needs_layout_passes=False),
        scratch_types=[
            pltpu.VMEM((4, NP), i32),
            pltpu.VMEM((4, NP), i32),
            pltpu.VMEM((SB,), i32),
            pltpu.VMEM((SB,), i32),
            pltpu.VMEM((NP,), i32),
        ],
    )
    mh2T, hll1T, hll2T, degc = minmax_call(mhT, hllT, src, dst)

    gcn_call = pl.kernel(
        functools.partial(_sc_gcn_body, NP, NB),
        out_type=jax.ShapeDtypeStruct((H, NP), f32),
        mesh=mesh,
        compiler_params=pltpu.CompilerParams(needs_layout_passes=False),
        scratch_types=[
            pltpu.VMEM((4, NP), f32),
            pltpu.VMEM((4, NP), f32),
            pltpu.VMEM((SB,), i32),
            pltpu.VMEM((SB,), i32),
        ],
    )

    R = 2000
    G = N // R
    enc_call = pl.pallas_call(
        _tc_enc_body,
        grid=(G,),
        in_specs=[
            pl.BlockSpec((R, F), lambda i: (i, i * 0)),
            pl.BlockSpec((F, H), lambda i: (i * 0, i * 0)),
            pl.BlockSpec((1, H), lambda i: (i * 0, i * 0)),
        ],
        out_specs=pl.BlockSpec((R, H), lambda i: (i, i * 0)),
        out_shape=jax.ShapeDtypeStruct((N, H), f32),
    )
    h0 = enc_call(x, W_enc, b_enc.reshape(1, H))

    g1_call = pl.pallas_call(
        _tc_g1_body,
        grid=(G,),
        in_specs=[
            pl.BlockSpec((R, H), lambda i: (i, i * 0)),
            pl.BlockSpec((H, H), lambda i: (i * 0, i * 0)),
            pl.BlockSpec((R, 1), lambda i: (i, i * 0)),
        ],
        out_specs=[
            pl.BlockSpec((R, H), lambda i: (i, i * 0)),
            pl.BlockSpec((R, 1), lambda i: (i, i * 0)),
        ],
        out_shape=[
            jax.ShapeDtypeStruct((N, H), f32),
            jax.ShapeDtypeStruct((N, 1), f32),
        ],
    )
    g1, dv = g1_call(h0, W1, degc[:N].reshape(N, 1))

    acc1T = gcn_call(jnp.pad(g1.T, ((0, 0), (0, NP - N))), src, dst)
    acc1 = acc1T[:, :N].T

    layer_call = pl.pallas_call(
        _tc_layer_body,
        grid=(G,),
        in_specs=[
            pl.BlockSpec((R, H), lambda i: (i, i * 0)),
            pl.BlockSpec((R, H), lambda i: (i, i * 0)),
            pl.BlockSpec((R, H), lambda i: (i, i * 0)),
            pl.BlockSpec((R, 1), lambda i: (i, i * 0)),
            pl.BlockSpec((1, H), lambda i: (i * 0, i * 0)),
            pl.BlockSpec((H, H), lambda i: (i * 0, i * 0)),
        ],
        out_specs=[
            pl.BlockSpec((R, H), lambda i: (i, i * 0)),
            pl.BlockSpec((R, H), lambda i: (i, i * 0)),
        ],
        out_shape=[
            jax.ShapeDtypeStruct((N, H), f32),
            jax.ShapeDtypeStruct((N, H), f32),
        ],
    )
    h1, g2 = layer_call(h0, g1, acc1, dv, b1.reshape(1, H), W2)

    acc2T = gcn_call(jnp.pad(g2.T, ((0, 0), (0, NP - N))), src, dst)
    acc2 = acc2T[:, :N].T

    PADV = 200
    r1 = jnp.pad(hll1T[:, :N].T, ((0, 0), (0, 128 - M)), constant_values=PADV)
    r2 = jnp.pad(hll2T[:, :N].T, ((0, 0), (0, 128 - M)), constant_values=PADV)
    alpha_m2 = (0.7213 / (1.0 + 1.079 / M)) * M * M
    final_call = pl.pallas_call(
        functools.partial(_tc_final_body, alpha_m2),
        grid=(G,),
        in_specs=[
            pl.BlockSpec((R, H), lambda i: (i, i * 0)),
            pl.BlockSpec((R, H), lambda i: (i, i * 0)),
            pl.BlockSpec((R, H), lambda i: (i, i * 0)),
            pl.BlockSpec((R, 1), lambda i: (i, i * 0)),
            pl.BlockSpec((1, H), lambda i: (i * 0, i * 0)),
            pl.BlockSpec((R, 128), lambda i: (i, i * 0)),
            pl.BlockSpec((R, 128), lambda i: (i, i * 0)),
        ],
        out_specs=[
            pl.BlockSpec((R, H), lambda i: (i, i * 0)),
            pl.BlockSpec((R, 1), lambda i: (i, i * 0)),
            pl.BlockSpec((R, 1), lambda i: (i, i * 0)),
        ],
        out_shape=[
            jax.ShapeDtypeStruct((N, H), f32),
            jax.ShapeDtypeStruct((N, 1), f32),
            jax.ShapeDtypeStruct((N, 1), f32),
        ],
    )
    h2, c1, c2 = final_call(h1, g2, acc2, dv, b2.reshape(1, H), r1, r2)

    cards = jnp.concatenate([c1, c2], axis=1)
    minhash_out = mh2T[:, :N].T.astype(init_minhash.dtype)
    hll_out = hll2T[:, :N].T.astype(init_hll.dtype)
    return (h2, cards, minhash_out, hll_out)


# speedup vs baseline: 11.2005x; 1.2854x over previous
import functools

import jax
import jax.numpy as jnp
from jax import lax
from jax.experimental import pallas as pl
from jax.experimental.pallas import tpu as pltpu
from jax.experimental.pallas import tpu_sc as plsc

L = 16
SB = 4096
_LN2 = 0.6931471805599453


def _row_ids(r):
    return jnp.full((L,), r, jnp.int32)


def _scan_blocks(src_hbm, dst_hbm, nblocks, sbuf, dbuf, chunk_fn, unroll=1):
    def blk(b, carry):
        off = pl.multiple_of(b * jnp.int32(SB), SB)
        pltpu.sync_copy(src_hbm.at[pl.ds(off, SB)], sbuf)
        pltpu.sync_copy(dst_hbm.at[pl.ds(off, SB)], dbuf)

        def chunk(j, c2):
            coff = pl.multiple_of(j * jnp.int32(L * unroll), L)
            for u in range(unroll):
                s_idx = sbuf[pl.ds(coff + L * u, L)]
                d_idx = dbuf[pl.ds(coff + L * u, L)]
                chunk_fn(s_idx, d_idx)
            return c2
        return lax.fori_loop(jnp.int32(0), jnp.int32(SB // (L * unroll)),
                             chunk, carry)
    lax.fori_loop(jnp.int32(0), jnp.int32(nblocks), blk, jnp.int32(0))


def _copy_rows(src, dst, nrows, np_):
    def b(i, carry):
        off = pl.multiple_of(i * jnp.int32(L), L)
        for r in range(nrows):
            dst[r, pl.ds(off, L)] = src[r, pl.ds(off, L)]
        return carry
    lax.fori_loop(jnp.int32(0), jnp.int32(np_ // L), b, jnp.int32(0))


def _zero_rows(dst, nrows, np_):
    z = jnp.zeros((L,), dst.dtype)
    def b(i, carry):
        off = pl.multiple_of(i * jnp.int32(L), L)
        for r in range(nrows):
            dst[r, pl.ds(off, L)] = z
        return carry
    lax.fori_loop(jnp.int32(0), jnp.int32(np_ // L), b, jnp.int32(0))


def _gather16(v, idx):
    return v.at[idx].get(mode="promise_in_bounds")


def _minmax_chunk(old_ref, new_ref, rows, s_idx, d_idx, is_min):
    comb = jnp.minimum if is_min else jnp.maximum
    lane = lax.iota(jnp.int32, L)
    last = jnp.int32(L - 1)
    sk, perm = plsc.sort_key_val(d_idx, lane)
    nxt = jnp.minimum(lane + jnp.int32(1), last)
    dup = (sk == _gather16(sk, nxt)) & (lane < last)
    has_dup = jnp.max(dup.astype(jnp.int32)) > 0

    @pl.when(jnp.logical_not(has_dup))
    def _():
        for row in rows:
            sv = plsc.load_gather(old_ref, [row, s_idx])
            cur = plsc.load_gather(new_ref, [row, d_idx])
            plsc.store_scatter(new_ref, [row, d_idx], comb(cur, sv))

    @pl.when(has_dup)
    def _():
        sames, idxs = [], []
        for d in (1, 2, 4, 8):
            idx_d = jnp.minimum(lane + jnp.int32(d), last)
            inb = (lane + jnp.int32(d)) < jnp.int32(L)
            sames.append(inb & (sk == _gather16(sk, idx_d)))
            idxs.append(idx_d)
        prv = jnp.maximum(lane - jnp.int32(1), jnp.int32(0))
        is_first = (lane == 0) | (sk != _gather16(sk, prv))
        for row in rows:
            sv = plsc.load_gather(old_ref, [row, s_idx])
            vp = _gather16(sv, perm)
            for same, idx_d in zip(sames, idxs):
                vp = jnp.where(same, comb(vp, _gather16(vp, idx_d)), vp)
            cur = plsc.load_gather(new_ref, [row, sk])
            plsc.store_scatter(new_ref, [row, sk], comb(cur, vp),
                               mask=is_first)


def _sc_minmax_body(np_, nblocks,
                    mhT, hllT, srcE, dstE,
                    mh2T, hll1T, hll2T, degc,
                    old4, new4, sbuf, dbuf, degt):
    wid = lax.axis_index("s") * jnp.int32(2) + lax.axis_index("c")
    rows = [_row_ids(r) for r in range(4)]

    base = wid * jnp.int32(4)
    pltpu.sync_copy(mhT.at[pl.ds(base, 4)], old4)
    pltpu.sync_copy(mhT.at[pl.ds(base, 4)], new4)

    def mh_chunk(s_idx, d_idx):
        _minmax_chunk(old4, new4, rows, s_idx, d_idx, is_min=True)

    _scan_blocks(srcE, dstE, nblocks, sbuf, dbuf, mh_chunk)
    _copy_rows(new4, old4, 4, np_)
    _scan_blocks(srcE, dstE, nblocks, sbuf, dbuf, mh_chunk)
    pltpu.sync_copy(new4, mh2T.at[pl.ds(base, 4)])

    @pl.when(wid < 10)
    def _():
        pltpu.sync_copy(hllT.at[pl.ds(wid, 1)], old4.at[pl.ds(0, 1)])
        pltpu.sync_copy(hllT.at[pl.ds(wid, 1)], new4.at[pl.ds(0, 1)])

        def hl_chunk(s_idx, d_idx):
            _minmax_chunk(old4, new4, rows[:1], s_idx, d_idx, is_min=False)

        _scan_blocks(srcE, dstE, nblocks, sbuf, dbuf, hl_chunk)
        pltpu.sync_copy(new4.at[pl.ds(0, 1)], hll1T.at[pl.ds(wid, 1)])
        _copy_rows(new4, old4, 1, np_)
        _scan_blocks(srcE, dstE, nblocks, sbuf, dbuf, hl_chunk)
        pltpu.sync_copy(new4.at[pl.ds(0, 1)], hll2T.at[pl.ds(wid, 1)])

    @pl.when(wid == 10)
    def _():
        z = jnp.zeros((L,), jnp.int32)

        def zb(i, carry):
            degt[pl.ds(pl.multiple_of(i * jnp.int32(L), L), L)] = z
            return carry
        lax.fori_loop(jnp.int32(0), jnp.int32(np_ // L), zb, jnp.int32(0))

        ones = jnp.ones((L,), jnp.int32)

        def dg_chunk(s_idx, d_idx):
            plsc.addupdate_scatter(degt, [d_idx], ones)

        _scan_blocks(srcE, dstE, nblocks, sbuf, dbuf, dg_chunk, unroll=4)
        pltpu.sync_copy(degt, degc)


def _sc_gcn_body(np_, nblocks, gT, srcE, dstE, accT, val4, acc4, sbuf, dbuf):
    wid = lax.axis_index("s") * jnp.int32(2) + lax.axis_index("c")
    rows = [_row_ids(r) for r in range(4)]
    for p in range(2):
        base = wid * jnp.int32(8) + jnp.int32(p * 4)
        pltpu.sync_copy(gT.at[pl.ds(base, 4)], val4)
        _zero_rows(acc4, 4, np_)

        def chunk(s_idx, d_idx):
            for r in range(4):
                v = plsc.load_gather(val4, [rows[r], s_idx])
                plsc.addupdate_scatter(acc4, [rows[r], d_idx], v)

        _scan_blocks(srcE, dstE, nblocks, sbuf, dbuf, chunk, unroll=4)
        pltpu.sync_copy(acc4, accT.at[pl.ds(base, 4)])


def _tc_enc_body(xb, Wb, bb, ob):
    ob[...] = jnp.dot(xb[...], Wb[...],
                      preferred_element_type=jnp.float32) + bb[...]


def _tc_g1_body(hb, Wb, degb, gb, dvb):
    dinv = lax.rsqrt(degb[...].astype(jnp.float32) + 1.0)
    dvb[...] = dinv
    gb[...] = dinv * jnp.dot(hb[...], Wb[...],
                             preferred_element_type=jnp.float32)


def _tc_layer_body(hb, gb, ab, dvb, bb, Wb, h1b, g2b):
    h1 = hb[...] + dvb[...] * (ab[...] + gb[...]) + bb[...]
    h1b[...] = h1
    g2b[...] = dvb[...] * jnp.dot(h1, Wb[...],
                                  preferred_element_type=jnp.float32)


def _tc_final_body(alpha_m2, hb, gb, ab, dvb, bb, r1b, r2b,
                   hob, c1b, c2b):
    hob[...] = hb[...] + dvb[...] * (ab[...] + gb[...]) + bb[...]
    z1 = jnp.sum(jnp.exp(r1b[...].astype(jnp.float32) * (-_LN2)),
                 axis=1, keepdims=True)
    z2 = jnp.sum(jnp.exp(r2b[...].astype(jnp.float32) * (-_LN2)),
                 axis=1, keepdims=True)
    c1b[...] = alpha_m2 / z1
    c2b[...] = alpha_m2 / z2


def kernel(x, edge_index, init_minhash, init_hll, W_enc, b_enc, W1, b1, W2, b2):
    N, F = x.shape
    H = W_enc.shape[1]
    E = edge_index.shape[1]
    NPERM = init_minhash.shape[1]
    M = init_hll.shape[1]
    NP = ((N + L) // L) * L
    NB = -(-E // SB)
    EP = NB * SB

    i32 = jnp.int32
    f32 = jnp.float32

    src = jnp.concatenate(
        [edge_index[0].astype(i32), jnp.full((EP - E,), N, i32)])
    dst = jnp.concatenate(
        [edge_index[1].astype(i32), jnp.full((EP - E,), N, i32)])
    mhT = jnp.pad(init_minhash.astype(i32).T, ((0, 0), (0, NP - N)))
    hllT = jnp.pad(init_hll.astype(i32).T, ((0, 0), (0, NP - N)))

    mesh = plsc.VectorSubcoreMesh(core_axis_name="c", subcore_axis_name="s")

    minmax_call = pl.kernel(
        functools.partial(_sc_minmax_body, NP, NB),
        out_type=[
            jax.ShapeDtypeStruct((NPERM, NP), i32),
            jax.ShapeDtypeStruct((M, NP), i32),
            jax.ShapeDtypeStruct((M, NP), i32),
            jax.ShapeDtypeStruct((NP,), i32),
        ],
        mesh=mesh,
        compiler_params=pltpu.CompilerParams(needs_layout_passes=False),
        scratch_types=[
            pltpu.VMEM((4, NP), i32),
            pltpu.VMEM((4, NP), i32),
            pltpu.VMEM((SB,), i32),
            pltpu.VMEM((SB,), i32),
            pltpu.VMEM((NP,), i32),
        ],
    )
    mh2T, hll1T, hll2T, degc = minmax_call(mhT, hllT, src, dst)

    gcn_call = pl.kernel(
        functools.partial(_sc_gcn_body, NP, NB),
        out_type=jax.ShapeDtypeStruct((H, NP), f32),
        mesh=mesh,
        compiler_params=pltpu.CompilerParams(needs_layout_passes=False),
        scratch_types=[
            pltpu.VMEM((4, NP), f32),
            pltpu.VMEM((4, NP), f32),
            pltpu.VMEM((SB,), i32),
            pltpu.VMEM((SB,), i32),
        ],
    )

    R = 2000
    G = N // R
    enc_call = pl.pallas_call(
        _tc_enc_body,
        grid=(G,),
        in_specs=[
            pl.BlockSpec((R, F), lambda i: (i, i * 0)),
            pl.BlockSpec((F, H), lambda i: (i * 0, i * 0)),
            pl.BlockSpec((1, H), lambda i: (i * 0, i * 0)),
        ],
        out_specs=pl.BlockSpec((R, H), lambda i: (i, i * 0)),
        out_shape=jax.ShapeDtypeStruct((N, H), f32),
    )
    h0 = enc_call(x, W_enc, b_enc.reshape(1, H))

    g1_call = pl.pallas_call(
        _tc_g1_body,
        grid=(G,),
        in_specs=[
            pl.BlockSpec((R, H), lambda i: (i, i * 0)),
            pl.BlockSpec((H, H), lambda i: (i * 0, i * 0)),
            pl.BlockSpec((R, 1), lambda i: (i, i * 0)),
        ],
        out_specs=[
            pl.BlockSpec((R, H), lambda i: (i, i * 0)),
            pl.BlockSpec((R, 1), lambda i: (i, i * 0)),
        ],
        out_shape=[
            jax.ShapeDtypeStruct((N, H), f32),
            jax.ShapeDtypeStruct((N, 1), f32),
        ],
    )
    g1, dv = g1_call(h0, W1, degc[:N].reshape(N, 1))

    acc1T = gcn_call(jnp.pad(g1.T, ((0, 0), (0, NP - N))), src, dst)
    acc1 = acc1T[:, :N].T

    layer_call = pl.pallas_call(
        _tc_layer_body,
        grid=(G,),
        in_specs=[
            pl.BlockSpec((R, H), lambda i: (i, i * 0)),
            pl.BlockSpec((R, H), lambda i: (i, i * 0)),
            pl.BlockSpec((R, H), lambda i: (i, i * 0)),
            pl.BlockSpec((R, 1), lambda i: (i, i * 0)),
            pl.BlockSpec((1, H), lambda i: (i * 0, i * 0)),
            pl.BlockSpec((H, H), lambda i: (i * 0, i * 0)),
        ],
        out_specs=[
            pl.BlockSpec((R, H), lambda i: (i, i * 0)),
            pl.BlockSpec((R, H), lambda i: (i, i * 0)),
        ],
        out_shape=[
            jax.ShapeDtypeStruct((N, H), f32),
            jax.ShapeDtypeStruct((N, H), f32),
        ],
    )
    h1, g2 = layer_call(h0, g1, acc1, dv, b1.reshape(1, H), W2)

    acc2T = gcn_call(jnp.pad(g2.T, ((0, 0), (0, NP - N))), src, dst)
    acc2 = acc2T[:, :N].T

    PADV = 200
    r1 = jnp.pad(hll1T[:, :N].T, ((0, 0), (0, 128 - M)), constant_values=PADV)
    r2 = jnp.pad(hll2T[:, :N].T, ((0, 0), (0, 128 - M)), constant_values=PADV)
    alpha_m2 = (0.7213 / (1.0 + 1.079 / M)) * M * M
    final_call = pl.pallas_call(
        functools.partial(_tc_final_body, alpha_m2),
        grid=(G,),
        in_specs=[
            pl.BlockSpec((R, H), lambda i: (i, i * 0)),
            pl.BlockSpec((R, H), lambda i: (i, i * 0)),
            pl.BlockSpec((R, H), lambda i: (i, i * 0)),
            pl.BlockSpec((R, 1), lambda i: (i, i * 0)),
            pl.BlockSpec((1, H), lambda i: (i * 0, i * 0)),
            pl.BlockSpec((R, 128), lambda i: (i, i * 0)),
            pl.BlockSpec((R, 128), lambda i: (i, i * 0)),
        ],
        out_specs=[
            pl.BlockSpec((R, H), lambda i: (i, i * 0)),
            pl.BlockSpec((R, 1), lambda i: (i, i * 0)),
            pl.BlockSpec((R, 1), lambda i: (i, i * 0)),
        ],
        out_shape=[
            jax.ShapeDtypeStruct((N, H), f32),
            jax.ShapeDtypeStruct((N, 1), f32),
            jax.ShapeDtypeStruct((N, 1), f32),
        ],
    )
    h2, c1, c2 = final_call(h1, g2, acc2, dv, b2.reshape(1, H), r1, r2)

    cards = jnp.concatenate([c1, c2], axis=1)
    minhash_out = mh2T[:, :N].T.astype(init_minhash.dtype)
    hll_out = hll2T[:, :N].T.astype(init_hll.dtype)
    return (h2, cards, minhash_out, hll_out)


# speedup vs baseline: 17.4122x; 1.5546x over previous
import functools

import jax
import jax.numpy as jnp
from jax import lax
from jax.experimental import pallas as pl
from jax.experimental.pallas import tpu as pltpu
from jax.experimental.pallas import tpu_sc as plsc

L = 16
SB = 4096
_LN2 = 0.6931471805599453


def _row_ids(r):
    return jnp.full((L,), r, jnp.int32)


def _scan_blocks(src_hbm, dst_hbm, nblocks, bufs, sems, chunk_fn, unroll=1):
    s0, d0, s1, d1 = bufs
    sem0, sem1 = sems

    def start(b, sbuf, dbuf, sem):
        off = pl.multiple_of(b * jnp.int32(SB), SB)
        pltpu.async_copy(src_hbm.at[pl.ds(off, SB)], sbuf, sem)
        pltpu.async_copy(dst_hbm.at[pl.ds(off, SB)], dbuf, sem)

    def wait(sbuf, dbuf, sem):
        pltpu.make_async_copy(src_hbm.at[pl.ds(0, SB)], sbuf, sem).wait()
        pltpu.make_async_copy(src_hbm.at[pl.ds(0, SB)], dbuf, sem).wait()

    def process(sbuf, dbuf):
        def chunk(j, c2):
            coff = pl.multiple_of(j * jnp.int32(L * unroll), L)
            for u in range(unroll):
                s_idx = sbuf[pl.ds(coff + L * u, L)]
                d_idx = dbuf[pl.ds(coff + L * u, L)]
                chunk_fn(s_idx, d_idx)
            return c2
        lax.fori_loop(jnp.int32(0), jnp.int32(SB // (L * unroll)), chunk,
                      jnp.int32(0))

    start(jnp.int32(0), s0, d0, sem0)

    def blk2(g, carry):
        b0 = g * jnp.int32(2)
        wait(s0, d0, sem0)
        start(b0 + jnp.int32(1), s1, d1, sem1)
        process(s0, d0)
        wait(s1, d1, sem1)
        start(b0 + jnp.int32(2), s0, d0, sem0)
        process(s1, d1)
        return carry
    lax.fori_loop(jnp.int32(0), jnp.int32(nblocks // 2), blk2, jnp.int32(0))
    wait(s0, d0, sem0)


def _copy_flat(src, dst, nwords):
    def b(i, carry):
        off = pl.multiple_of(i * jnp.int32(L), L)
        dst[pl.ds(off, L)] = src[pl.ds(off, L)]
        return carry
    lax.fori_loop(jnp.int32(0), jnp.int32(nwords // L), b, jnp.int32(0))


def _zero_flat(dst, nwords):
    z = jnp.zeros((L,), dst.dtype)
    def b(i, carry):
        dst[pl.ds(pl.multiple_of(i * jnp.int32(L), L), L)] = z
        return carry
    lax.fori_loop(jnp.int32(0), jnp.int32(nwords // L), b, jnp.int32(0))


def _gather16(v, idx):
    return v.at[idx].get(mode="promise_in_bounds")


def _minmax_chunk(old_ref, new_ref, rowoffs, s_idx, d_idx, is_min):
    comb = jnp.minimum if is_min else jnp.maximum
    lane = lax.iota(jnp.int32, L)
    last = jnp.int32(L - 1)
    sk, perm = plsc.sort_key_val(d_idx, lane)
    nxt = jnp.minimum(lane + jnp.int32(1), last)
    dup = (sk == _gather16(sk, nxt)) & (lane < last)
    has_dup = jnp.max(dup.astype(jnp.int32)) > 0

    @pl.when(jnp.logical_not(has_dup))
    def _():
        for off in rowoffs:
            sv = plsc.load_gather(old_ref, [s_idx + off])
            cur = plsc.load_gather(new_ref, [d_idx + off])
            plsc.store_scatter(new_ref, [d_idx + off], comb(cur, sv))

    @pl.when(has_dup)
    def _():
        sames, idxs = [], []
        for d in (1, 2, 4, 8):
            idx_d = jnp.minimum(lane + jnp.int32(d), last)
            inb = (lane + jnp.int32(d)) < jnp.int32(L)
            sames.append(inb & (sk == _gather16(sk, idx_d)))
            idxs.append(idx_d)
        prv = jnp.maximum(lane - jnp.int32(1), jnp.int32(0))
        is_first = (lane == 0) | (sk != _gather16(sk, prv))
        for off in rowoffs:
            sv = plsc.load_gather(old_ref, [s_idx + off])
            vp = _gather16(sv, perm)
            for same, idx_d in zip(sames, idxs):
                vp = jnp.where(same, comb(vp, _gather16(vp, idx_d)), vp)
            cur = plsc.load_gather(new_ref, [sk + off])
            plsc.store_scatter(new_ref, [sk + off], comb(cur, vp),
                               mask=is_first)


def _sc_minmax_body(np_, nblocks,
                    mhT, hllT, srcE, dstE,
                    mh2T, hll1T, hll2T, degc,
                    old5, new5, s0, d0, s1, d1, sem0, sem1):
    wid = lax.axis_index("s") * jnp.int32(2) + lax.axis_index("c")
    bufs = (s0, d0, s1, d1)
    sems = (sem0, sem1)

    @pl.when(wid < 26)
    def _():
        offs = [jnp.int32(r * np_) for r in range(5)]
        base = wid * jnp.int32(5 * np_)
        pltpu.sync_copy(mhT.at[pl.ds(base, 5 * np_)], old5)
        pltpu.sync_copy(mhT.at[pl.ds(base, 5 * np_)], new5)

        def mh_chunk(s_idx, d_idx):
            _minmax_chunk(old5, new5, offs, s_idx, d_idx, is_min=True)

        _scan_blocks(srcE, dstE, nblocks, bufs, sems, mh_chunk)
        _copy_flat(new5, old5, 5 * np_)
        _scan_blocks(srcE, dstE, nblocks, bufs, sems, mh_chunk)
        pltpu.sync_copy(new5, mh2T.at[pl.ds(base, 5 * np_)])

    @pl.when((wid >= 26) & (wid < 31))
    def _():
        offs = [jnp.int32(r * np_) for r in range(2)]
        hbase = (wid - jnp.int32(26)) * jnp.int32(2 * np_)
        pltpu.sync_copy(hllT.at[pl.ds(hbase, 2 * np_)], old5.at[pl.ds(0, 2 * np_)])
        pltpu.sync_copy(hllT.at[pl.ds(hbase, 2 * np_)], new5.at[pl.ds(0, 2 * np_)])

        def hl_chunk(s_idx, d_idx):
            _minmax_chunk(old5, new5, offs, s_idx, d_idx, is_min=False)

        _scan_blocks(srcE, dstE, nblocks, bufs, sems, hl_chunk)
        pltpu.sync_copy(new5.at[pl.ds(0, 2 * np_)], hll1T.at[pl.ds(hbase, 2 * np_)])
        _copy_flat(new5.at[pl.ds(0, 2 * np_)], old5.at[pl.ds(0, 2 * np_)], 2 * np_)
        _scan_blocks(srcE, dstE, nblocks, bufs, sems, hl_chunk)
        pltpu.sync_copy(new5.at[pl.ds(0, 2 * np_)], hll2T.at[pl.ds(hbase, 2 * np_)])

    @pl.when(wid == 31)
    def _():
        _zero_flat(old5.at[pl.ds(0, np_)], np_)
        ones = jnp.ones((L,), jnp.int32)

        def dg_chunk(s_idx, d_idx):
            plsc.addupdate_scatter(old5, [d_idx], ones)

        _scan_blocks(srcE, dstE, nblocks, bufs, sems, dg_chunk, unroll=4)
        pltpu.sync_copy(old5.at[pl.ds(0, np_)], degc)


def _sc_gcn_body(np_, nblocks, gT, srcE, dstE, accT,
                 val4, acc4, s0, d0, s1, d1, sem0, sem1):
    wid = lax.axis_index("s") * jnp.int32(2) + lax.axis_index("c")
    offs = [jnp.int32(r * np_) for r in range(4)]
    bufs = (s0, d0, s1, d1)
    sems = (sem0, sem1)
    for p in range(2):
        base = wid * jnp.int32(8 * np_) + jnp.int32(p * 4 * np_)
        pltpu.sync_copy(gT.at[pl.ds(base, 4 * np_)], val4)
        _zero_flat(acc4, 4 * np_)

        def chunk(s_idx, d_idx):
            for off in offs:
                v = plsc.load_gather(val4, [s_idx + off])
                plsc.addupdate_scatter(acc4, [d_idx + off], v)

        _scan_blocks(srcE, dstE, nblocks, bufs, sems, chunk, unroll=4)
        pltpu.sync_copy(acc4, accT.at[pl.ds(base, 4 * np_)])


def _tc_enc_body(xb, Wb, bb, ob):
    ob[...] = jnp.dot(xb[...], Wb[...],
                      preferred_element_type=jnp.float32) + bb[...]


def _tc_g1_body(hb, Wb, degb, gb, dvb):
    dinv = lax.rsqrt(degb[...].astype(jnp.float32) + 1.0)
    dvb[...] = dinv
    gb[...] = dinv * jnp.dot(hb[...], Wb[...],
                             preferred_element_type=jnp.float32)


def _tc_layer_body(hb, gb, ab, dvb, bb, Wb, h1b, g2b):
    h1 = hb[...] + dvb[...] * (ab[...] + gb[...]) + bb[...]
    h1b[...] = h1
    g2b[...] = dvb[...] * jnp.dot(h1, Wb[...],
                                  preferred_element_type=jnp.float32)


def _tc_final_body(alpha_m2, hb, gb, ab, dvb, bb, r1b, r2b,
                   hob, c1b, c2b):
    hob[...] = hb[...] + dvb[...] * (ab[...] + gb[...]) + bb[...]
    z1 = jnp.sum(jnp.exp(r1b[...].astype(jnp.float32) * (-_LN2)),
                 axis=1, keepdims=True)
    z2 = jnp.sum(jnp.exp(r2b[...].astype(jnp.float32) * (-_LN2)),
                 axis=1, keepdims=True)
    c1b[...] = alpha_m2 / z1
    c2b[...] = alpha_m2 / z2


def kernel(x, edge_index, init_minhash, init_hll, W_enc, b_enc, W1, b1, W2, b2):
    N, F = x.shape
    H = W_enc.shape[1]
    E = edge_index.shape[1]
    NPERM = init_minhash.shape[1]
    M = init_hll.shape[1]
    NP = ((N + L) // L) * L
    NB = 2 * (-(-E // (2 * SB)))
    EP = (NB + 1) * SB
    MHR = 5 * 26

    i32 = jnp.int32
    f32 = jnp.float32

    src = jnp.concatenate(
        [edge_index[0].astype(i32), jnp.full((EP - E,), N, i32)])
    dst = jnp.concatenate(
        [edge_index[1].astype(i32), jnp.full((EP - E,), N, i32)])
    mhT = jnp.pad(init_minhash.astype(i32).T,
                  ((0, MHR - NPERM), (0, NP - N)))
    hllT = jnp.pad(init_hll.astype(i32).T, ((0, 0), (0, NP - N)))

    mesh = plsc.VectorSubcoreMesh(core_axis_name="c", subcore_axis_name="s")

    minmax_call = pl.kernel(
        functools.partial(_sc_minmax_body, NP, NB),
        out_type=[
            jax.ShapeDtypeStruct((MHR * NP,), i32),
            jax.ShapeDtypeStruct((M * NP,), i32),
            jax.ShapeDtypeStruct((M * NP,), i32),
            jax.ShapeDtypeStruct((NP,), i32),
        ],
        mesh=mesh,
        compiler_params=pltpu.CompilerParams(needs_layout_passes=False),
        scratch_types=[
            pltpu.VMEM((5 * NP,), i32),
            pltpu.VMEM((5 * NP,), i32),
            pltpu.VMEM((SB,), i32),
            pltpu.VMEM((SB,), i32),
            pltpu.VMEM((SB,), i32),
            pltpu.VMEM((SB,), i32),
            pltpu.SemaphoreType.DMA,
            pltpu.SemaphoreType.DMA,
        ],
    )
    mh2T, hll1T, hll2T, degc = minmax_call(
        mhT.reshape(-1), hllT.reshape(-1), src, dst)
    mh2T = mh2T.reshape(MHR, NP)
    hll1T = hll1T.reshape(M, NP)
    hll2T = hll2T.reshape(M, NP)

    gcn_call = pl.kernel(
        functools.partial(_sc_gcn_body, NP, NB),
        out_type=jax.ShapeDtypeStruct((H * NP,), f32),
        mesh=mesh,
        compiler_params=pltpu.CompilerParams(needs_layout_passes=False),
        scratch_types=[
            pltpu.VMEM((4 * NP,), f32),
            pltpu.VMEM((4 * NP,), f32),
            pltpu.VMEM((SB,), i32),
            pltpu.VMEM((SB,), i32),
            pltpu.VMEM((SB,), i32),
            pltpu.VMEM((SB,), i32),
            pltpu.SemaphoreType.DMA,
            pltpu.SemaphoreType.DMA,
        ],
    )

    R = 2000
    G = N // R
    enc_call = pl.pallas_call(
        _tc_enc_body,
        grid=(G,),
        in_specs=[
            pl.BlockSpec((R, F), lambda i: (i, i * 0)),
            pl.BlockSpec((F, H), lambda i: (i * 0, i * 0)),
            pl.BlockSpec((1, H), lambda i: (i * 0, i * 0)),
        ],
        out_specs=pl.BlockSpec((R, H), lambda i: (i, i * 0)),
        out_shape=jax.ShapeDtypeStruct((N, H), f32),
    )
    h0 = enc_call(x, W_enc, b_enc.reshape(1, H))

    g1_call = pl.pallas_call(
        _tc_g1_body,
        grid=(G,),
        in_specs=[
            pl.BlockSpec((R, H), lambda i: (i, i * 0)),
            pl.BlockSpec((H, H), lambda i: (i * 0, i * 0)),
            pl.BlockSpec((R, 1), lambda i: (i, i * 0)),
        ],
        out_specs=[
            pl.BlockSpec((R, H), lambda i: (i, i * 0)),
            pl.BlockSpec((R, 1), lambda i: (i, i * 0)),
        ],
        out_shape=[
            jax.ShapeDtypeStruct((N, H), f32),
            jax.ShapeDtypeStruct((N, 1), f32),
        ],
    )
    g1, dv = g1_call(h0, W1, degc[:N].reshape(N, 1))

    acc1T = gcn_call(jnp.pad(g1.T, ((0, 0), (0, NP - N))).reshape(-1),
                     src, dst).reshape(H, NP)
    acc1 = acc1T[:, :N].T

    layer_call = pl.pallas_call(
        _tc_layer_body,
        grid=(G,),
        in_specs=[
            pl.BlockSpec((R, H), lambda i: (i, i * 0)),
            pl.BlockSpec((R, H), lambda i: (i, i * 0)),
            pl.BlockSpec((R, H), lambda i: (i, i * 0)),
            pl.BlockSpec((R, 1), lambda i: (i, i * 0)),
            pl.BlockSpec((1, H), lambda i: (i * 0, i * 0)),
            pl.BlockSpec((H, H), lambda i: (i * 0, i * 0)),
        ],
        out_specs=[
            pl.BlockSpec((R, H), lambda i: (i, i * 0)),
            pl.BlockSpec((R, H), lambda i: (i, i * 0)),
        ],
        out_shape=[
            jax.ShapeDtypeStruct((N, H), f32),
            jax.ShapeDtypeStruct((N, H), f32),
        ],
    )
    h1, g2 = layer_call(h0, g1, acc1, dv, b1.reshape(1, H), W2)

    acc2T = gcn_call(jnp.pad(g2.T, ((0, 0), (0, NP - N))).reshape(-1),
                     src, dst).reshape(H, NP)
    acc2 = acc2T[:, :N].T

    PADV = 200
    r1 = jnp.pad(hll1T[:, :N].T, ((0, 0), (0, 128 - M)), constant_values=PADV)
    r2 = jnp.pad(hll2T[:, :N].T, ((0, 0), (0, 128 - M)), constant_values=PADV)
    alpha_m2 = (0.7213 / (1.0 + 1.079 / M)) * M * M
    final_call = pl.pallas_call(
        functools.partial(_tc_final_body, alpha_m2),
        grid=(G,),
        in_specs=[
            pl.BlockSpec((R, H), lambda i: (i, i * 0)),
            pl.BlockSpec((R, H), lambda i: (i, i * 0)),
            pl.BlockSpec((R, H), lambda i: (i, i * 0)),
            pl.BlockSpec((R, 1), lambda i: (i, i * 0)),
            pl.BlockSpec((1, H), lambda i: (i * 0, i * 0)),
            pl.BlockSpec((R, 128), lambda i: (i, i * 0)),
            pl.BlockSpec((R, 128), lambda i: (i, i * 0)),
        ],
        out_specs=[
            pl.BlockSpec((R, H), lambda i: (i, i * 0)),
            pl.BlockSpec((R, 1), lambda i: (i, i * 0)),
            pl.BlockSpec((R, 1), lambda i: (i, i * 0)),
        ],
        out_shape=[
            jax.ShapeDtypeStruct((N, H), f32),
            jax.ShapeDtypeStruct((N, 1), f32),
            jax.ShapeDtypeStruct((N, 1), f32),
        ],
    )
    h2, c1, c2 = final_call(h1, g2, acc2, dv, b2.reshape(1, H), r1, r2)

    cards = jnp.concatenate([c1, c2], axis=1)
    minhash_out = mh2T[:NPERM, :N].T.astype(init_minhash.dtype)
    hll_out = hll2T[:, :N].T.astype(init_hll.dtype)
    return (h2, cards, minhash_out, hll_out)
